# baseline (speedup 1.0000x reference)
import jax
import jax.numpy as jnp
from jax import lax
from jax.experimental import pallas as pl
from jax.experimental.pallas import tpu as pltpu
from jax.experimental.pallas import tpu_sc as plsc

N = 10000
E = 320000
D = 128
G = 128
NC = 2
NS = 16
CW = D // NS
CW0 = 2 * CW
EH = E // NC
C = 80
NIB = 8
NCB = EH // (C * NIB)
_B = 1000
_NBLK = N // _B

_PREC = lax.Precision.HIGHEST


def _make_sc_edge():
  mesh = plsc.VectorSubcoreMesh(core_axis_name="c", subcore_axis_name="s",
                                num_cores=NC, num_subcores=NS)
  out_type = (jax.ShapeDtypeStruct((NC, NS - 1, N, CW), jnp.float32),
              jax.ShapeDtypeStruct((NC, N, CW0), jnp.float32))
  scratch = [
      pltpu.VMEM((NCB, C), jnp.int32),
      pltpu.VMEM((NCB, C), jnp.int32),
      pltpu.VMEM((C, CW), jnp.float32),
      pltpu.VMEM((C, CW0), jnp.float32),
      pltpu.VMEM_SHARED((NS - 1, N, CW), jnp.float32),
      pltpu.VMEM_SHARED((N, CW0), jnp.float32),
      pltpu.SemaphoreType.DMA,
  ]

  def body(y_hbm, y0_hbm, src_hbm, dst_hbm, z_hbm, z0_hbm,
           part_hbm, part0_hbm, sidx, didx, rows, rows0, acc, acc0, sem):
    c = lax.axis_index("c")
    s = lax.axis_index("s")

    @pl.when(s == 0)
    def _():
      pltpu.sync_copy(z0_hbm, acc0)

    @pl.when(s > 0)
    def _():
      pltpu.sync_copy(z_hbm, acc.at[s - 1])

    def block(b, carry):
      pltpu.sync_copy(src_hbm.at[c, b], sidx)
      pltpu.sync_copy(dst_hbm.at[c, b], didx)

      def chunk(k, carry2):
        @pl.when(s == 0)
        def _():
          pltpu.async_copy(y0_hbm.at[sidx.at[k]], rows0, sem).wait()
          pltpu.sync_copy(rows0, acc0.at[didx.at[k]], add=True)

        @pl.when(s > 0)
        def _():
          pltpu.async_copy(y_hbm.at[s - 1].at[sidx.at[k]], rows, sem).wait()
          pltpu.sync_copy(rows, acc.at[s - 1].at[didx.at[k]], add=True)
        return carry2

      lax.fori_loop(0, NCB, chunk, 0)
      return carry

    lax.fori_loop(0, NIB, block, 0)

    @pl.when(s == 0)
    def _():
      pltpu.sync_copy(acc0, part0_hbm.at[c])

    @pl.when(s > 0)
    def _():
      pltpu.sync_copy(acc.at[s - 1], part_hbm.at[c, s - 1])

  return pl.kernel(body, out_type=out_type, mesh=mesh, scratch_types=scratch,
                   compiler_params=pltpu.CompilerParams(
                       use_tc_tiling_on_sc=False))


def _tc_layer(part, part0, h, Wl, Wr, b, *, relu):
  dw_out = Wr.shape[-1]
  hw = h.shape[-1]

  def body(part_ref, part0_ref, h_ref, Wl_ref, Wr_ref, b_ref, out_ref):
    p0 = part0_ref[0] + part0_ref[1]
    deg = jnp.maximum(p0[:, CW], 1.0)
    acc = jnp.dot(p0[:, :CW], Wl_ref[0:CW, :], precision=_PREC,
                  preferred_element_type=jnp.float32)
    for s in range(1, NS):
      psum = part_ref[0, s - 1] + part_ref[1, s - 1]
      acc = acc + jnp.dot(psum, Wl_ref[s * CW:(s + 1) * CW, :],
                          precision=_PREC,
                          preferred_element_type=jnp.float32)
    acc = acc / deg[:, None]
    acc = acc + jnp.dot(h_ref[...], Wr_ref[...], precision=_PREC,
                        preferred_element_type=jnp.float32) + b_ref[...]
    if relu:
      acc = jnp.maximum(acc, 0.0)
    out_ref[...] = acc

  return pl.pallas_call(
      body,
      grid=(_NBLK,),
      in_specs=[
          pl.BlockSpec((NC, NS - 1, _B, CW), lambda i: (0, 0, i, 0)),
          pl.BlockSpec((NC, _B, CW0), lambda i: (0, i, 0)),
          pl.BlockSpec((_B, hw), lambda i: (i, 0)),
          pl.BlockSpec(Wl.shape, lambda i: (0, 0)),
          pl.BlockSpec(Wr.shape, lambda i: (0, 0)),
          pl.BlockSpec((1, dw_out), lambda i: (0, 0)),
      ],
      out_specs=pl.BlockSpec((_B, dw_out), lambda i: (i, 0)),
      out_shape=jax.ShapeDtypeStruct((N, dw_out), jnp.float32),
  )(part, part0, h, Wl, Wr, b.reshape(1, dw_out))


def _tc_pool(h, batch3, Wlin, blin):
  dw = h.shape[-1]
  ow = Wlin.shape[-1]

  def body(h_ref, b_ref, Wlin_ref, blin_ref, out_ref, acc_s, acc_c):
    i = pl.program_id(0)

    @pl.when(i == 0)
    def _():
      acc_s[...] = jnp.zeros_like(acc_s)
      acc_c[...] = jnp.zeros_like(acc_c)

    bv = b_ref[0, 0, :]
    gi = lax.broadcasted_iota(jnp.int32, (G, _B), 0)
    mask = (bv[None, :] == gi).astype(jnp.float32)
    acc_s[...] += jnp.dot(mask, h_ref[...], precision=_PREC,
                          preferred_element_type=jnp.float32)
    acc_c[...] += jnp.sum(mask, axis=1, keepdims=True)

    @pl.when(i == _NBLK - 1)
    def _():
      pooled = acc_s[...] / jnp.maximum(acc_c[...], 1.0)
      out_ref[...] = jnp.dot(pooled, Wlin_ref[...], precision=_PREC,
                             preferred_element_type=jnp.float32) + blin_ref[...]

  return pl.pallas_call(
      body,
      grid=(_NBLK,),
      in_specs=[
          pl.BlockSpec((_B, dw), lambda i: (i, 0)),
          pl.BlockSpec((1, 1, _B), lambda i: (i, 0, 0)),
          pl.BlockSpec(Wlin.shape, lambda i: (0, 0)),
          pl.BlockSpec((1, ow), lambda i: (0, 0)),
      ],
      out_specs=pl.BlockSpec((G, ow), lambda i: (0, 0)),
      out_shape=jax.ShapeDtypeStruct((G, ow), jnp.float32),
      scratch_shapes=[pltpu.VMEM((G, dw), jnp.float32),
                      pltpu.VMEM((G, 1), jnp.float32)],
  )(h, batch3, Wlin, blin.reshape(1, ow))


def kernel(x, edge_index, batch, Wl1, Wr1, b1, Wl2, Wr2, b2,
           Wl3, Wr3, b3, Wlin, blin):
  src = edge_index[0].astype(jnp.int32).reshape(NC, NIB, NCB, C)
  dst = edge_index[1].astype(jnp.int32).reshape(NC, NIB, NCB, C)
  batch3 = batch.astype(jnp.int32).reshape(_NBLK, 1, _B)
  z = jnp.zeros((N, CW), jnp.float32)
  z0 = jnp.zeros((N, CW0), jnp.float32)
  onecol = jnp.concatenate([jnp.ones((N, 1), jnp.float32),
                            jnp.zeros((N, CW - 1), jnp.float32)], axis=1)

  sc = _make_sc_edge()

  def cols(y):
    return y.reshape(N, NS, CW).transpose(1, 0, 2)[1:]

  def aug0(y):
    return jnp.concatenate([y[:, :CW], onecol], axis=1)

  part1, p01 = sc(cols(x), aug0(x), src, dst, z, z0)
  h1 = _tc_layer(part1, p01, x, Wl1, Wr1, b1, relu=True)
  part2, p02 = sc(cols(h1), aug0(h1), src, dst, z, z0)
  h2 = _tc_layer(part2, p02, h1, Wl2, Wr2, b2, relu=True)
  part3, p03 = sc(cols(h2), aug0(h2), src, dst, z, z0)
  h3 = _tc_layer(part3, p03, h2, Wl3, Wr3, b3, relu=False)
  return _tc_pool(h3, batch3, Wlin, blin)


# speedup vs baseline: 2.0006x; 2.0006x over previous
import jax
import jax.numpy as jnp
from jax import lax
from jax.experimental import pallas as pl
from jax.experimental.pallas import tpu as pltpu
from jax.experimental.pallas import tpu_sc as plsc

N = 10000
E = 320000
D = 128
G = 128
NC = 2
NS = 16
CW = D // NS
CW0 = 2 * CW
EH = E // NC
C = 128
NIB = 10
NCB = EH // (C * NIB)
_B = 1000
_NBLK = N // _B

_PREC = lax.Precision.HIGHEST


def _make_sc_edge():
  mesh = plsc.VectorSubcoreMesh(core_axis_name="c", subcore_axis_name="s",
                                num_cores=NC, num_subcores=NS)
  out_type = (jax.ShapeDtypeStruct((NC, NS - 1, N, CW), jnp.float32),
              jax.ShapeDtypeStruct((NC, N, CW0), jnp.float32))
  scratch = [
      pltpu.VMEM((NCB, C), jnp.int32),
      pltpu.VMEM((NCB, C), jnp.int32),
      pltpu.VMEM((2, C, CW), jnp.float32),
      pltpu.VMEM((2, C, CW0), jnp.float32),
      pltpu.VMEM_SHARED((NS - 1, N, CW), jnp.float32),
      pltpu.VMEM_SHARED((N, CW0), jnp.float32),
      (pltpu.SemaphoreType.DMA, pltpu.SemaphoreType.DMA),
      (pltpu.SemaphoreType.DMA, pltpu.SemaphoreType.DMA),
  ]

  def body(y_hbm, y0_hbm, src_hbm, dst_hbm, z_hbm, z0_hbm,
           part_hbm, part0_hbm, sidx, didx, rows, rows0, acc, acc0,
           gsem, ssem):
    c = lax.axis_index("c")
    s = lax.axis_index("s")

    @pl.when(s == 0)
    def _():
      pltpu.sync_copy(z0_hbm, acc0)

    @pl.when(s > 0)
    def _():
      pltpu.sync_copy(z_hbm, acc.at[s - 1])

    myacc = (acc0, acc.at[s - 1])
    ybase = (y0_hbm, y_hbm.at[s - 1])
    rbuf = (rows0, rows)

    def gissue(k, p):
      @pl.when(s == 0)
      def _():
        pltpu.async_copy(ybase[0].at[sidx.at[k]], rbuf[0].at[p], gsem[p])

      @pl.when(s > 0)
      def _():
        pltpu.async_copy(ybase[1].at[sidx.at[k]], rbuf[1].at[p], gsem[p])

    def gwait(k, p):
      @pl.when(s == 0)
      def _():
        pltpu.make_async_copy(ybase[0].at[sidx.at[k]], rbuf[0].at[p],
                              gsem[p]).wait()

      @pl.when(s > 0)
      def _():
        pltpu.make_async_copy(ybase[1].at[sidx.at[k]], rbuf[1].at[p],
                              gsem[p]).wait()

    def sissue(k, p):
      @pl.when(s == 0)
      def _():
        pltpu.async_copy(rbuf[0].at[p], myacc[0].at[didx.at[k]], ssem[p],
                         add=True)

      @pl.when(s > 0)
      def _():
        pltpu.async_copy(rbuf[1].at[p], myacc[1].at[didx.at[k]], ssem[p],
                         add=True)

    def swait(k, p):
      @pl.when(s == 0)
      def _():
        pltpu.make_async_copy(rbuf[0].at[p], myacc[0].at[didx.at[k]],
                              ssem[p]).wait()

      @pl.when(s > 0)
      def _():
        pltpu.make_async_copy(rbuf[1].at[p], myacc[1].at[didx.at[k]],
                              ssem[p]).wait()

    def block(b, carry):
      pltpu.sync_copy(src_hbm.at[c, b], sidx)
      pltpu.sync_copy(dst_hbm.at[c, b], didx)

      gissue(0, 0)

      def chunk(k, carry2):
        for p in range(2):
          @pl.when(lax.rem(k, 2) == p)
          def _():
            @pl.when(k + 1 < NCB)
            def _():
              @pl.when(k >= 1)
              def _():
                swait(k - 1, 1 - p)
              gissue(k + 1, 1 - p)
            gwait(k, p)
            sissue(k, p)
        return carry2

      lax.fori_loop(0, NCB, chunk, 0)
      swait(NCB - 2, (NCB - 2) % 2)
      swait(NCB - 1, (NCB - 1) % 2)
      return carry

    lax.fori_loop(0, NIB, block, 0)

    @pl.when(s == 0)
    def _():
      pltpu.sync_copy(acc0, part0_hbm.at[c])

    @pl.when(s > 0)
    def _():
      pltpu.sync_copy(acc.at[s - 1], part_hbm.at[c, s - 1])

  return pl.kernel(body, out_type=out_type, mesh=mesh, scratch_types=scratch,
                   compiler_params=pltpu.CompilerParams(
                       use_tc_tiling_on_sc=False))


def _tc_layer(part, part0, h, Wl, Wr, b, *, relu):
  dw_out = Wr.shape[-1]
  hw = h.shape[-1]

  def body(part_ref, part0_ref, h_ref, Wl_ref, Wr_ref, b_ref, out_ref):
    p0 = part0_ref[0] + part0_ref[1]
    deg = jnp.maximum(p0[:, CW], 1.0)
    acc = jnp.dot(p0[:, :CW], Wl_ref[0:CW, :], precision=_PREC,
                  preferred_element_type=jnp.float32)
    for s in range(1, NS):
      psum = part_ref[0, s - 1] + part_ref[1, s - 1]
      acc = acc + jnp.dot(psum, Wl_ref[s * CW:(s + 1) * CW, :],
                          precision=_PREC,
                          preferred_element_type=jnp.float32)
    acc = acc / deg[:, None]
    acc = acc + jnp.dot(h_ref[...], Wr_ref[...], precision=_PREC,
                        preferred_element_type=jnp.float32) + b_ref[...]
    if relu:
      acc = jnp.maximum(acc, 0.0)
    out_ref[...] = acc

  return pl.pallas_call(
      body,
      grid=(_NBLK,),
      in_specs=[
          pl.BlockSpec((NC, NS - 1, _B, CW), lambda i: (0, 0, i, 0)),
          pl.BlockSpec((NC, _B, CW0), lambda i: (0, i, 0)),
          pl.BlockSpec((_B, hw), lambda i: (i, 0)),
          pl.BlockSpec(Wl.shape, lambda i: (0, 0)),
          pl.BlockSpec(Wr.shape, lambda i: (0, 0)),
          pl.BlockSpec((1, dw_out), lambda i: (0, 0)),
      ],
      out_specs=pl.BlockSpec((_B, dw_out), lambda i: (i, 0)),
      out_shape=jax.ShapeDtypeStruct((N, dw_out), jnp.float32),
  )(part, part0, h, Wl, Wr, b.reshape(1, dw_out))


def _tc_pool(h, batch3, Wlin, blin):
  dw = h.shape[-1]
  ow = Wlin.shape[-1]

  def body(h_ref, b_ref, Wlin_ref, blin_ref, out_ref, acc_s, acc_c):
    i = pl.program_id(0)

    @pl.when(i == 0)
    def _():
      acc_s[...] = jnp.zeros_like(acc_s)
      acc_c[...] = jnp.zeros_like(acc_c)

    bv = b_ref[0, 0, :]
    gi = lax.broadcasted_iota(jnp.int32, (G, _B), 0)
    mask = (bv[None, :] == gi).astype(jnp.float32)
    acc_s[...] += jnp.dot(mask, h_ref[...], precision=_PREC,
                          preferred_element_type=jnp.float32)
    acc_c[...] += jnp.sum(mask, axis=1, keepdims=True)

    @pl.when(i == _NBLK - 1)
    def _():
      pooled = acc_s[...] / jnp.maximum(acc_c[...], 1.0)
      out_ref[...] = jnp.dot(pooled, Wlin_ref[...], precision=_PREC,
                             preferred_element_type=jnp.float32) + blin_ref[...]

  return pl.pallas_call(
      body,
      grid=(_NBLK,),
      in_specs=[
          pl.BlockSpec((_B, dw), lambda i: (i, 0)),
          pl.BlockSpec((1, 1, _B), lambda i: (i, 0, 0)),
          pl.BlockSpec(Wlin.shape, lambda i: (0, 0)),
          pl.BlockSpec((1, ow), lambda i: (0, 0)),
      ],
      out_specs=pl.BlockSpec((G, ow), lambda i: (0, 0)),
      out_shape=jax.ShapeDtypeStruct((G, ow), jnp.float32),
      scratch_shapes=[pltpu.VMEM((G, dw), jnp.float32),
                      pltpu.VMEM((G, 1), jnp.float32)],
  )(h, batch3, Wlin, blin.reshape(1, ow))


def kernel(x, edge_index, batch, Wl1, Wr1, b1, Wl2, Wr2, b2,
           Wl3, Wr3, b3, Wlin, blin):
  src = edge_index[0].astype(jnp.int32).reshape(NC, NIB, NCB, C)
  dst = edge_index[1].astype(jnp.int32).reshape(NC, NIB, NCB, C)
  batch3 = batch.astype(jnp.int32).reshape(_NBLK, 1, _B)
  z = jnp.zeros((N, CW), jnp.float32)
  z0 = jnp.zeros((N, CW0), jnp.float32)
  onecol = jnp.concatenate([jnp.ones((N, 1), jnp.float32),
                            jnp.zeros((N, CW - 1), jnp.float32)], axis=1)

  sc = _make_sc_edge()

  def cols(y):
    return y.reshape(N, NS, CW).transpose(1, 0, 2)[1:]

  def aug0(y):
    return jnp.concatenate([y[:, :CW], onecol], axis=1)

  part1, p01 = sc(cols(x), aug0(x), src, dst, z, z0)
  h1 = _tc_layer(part1, p01, x, Wl1, Wr1, b1, relu=True)
  part2, p02 = sc(cols(h1), aug0(h1), src, dst, z, z0)
  h2 = _tc_layer(part2, p02, h1, Wl2, Wr2, b2, relu=True)
  part3, p03 = sc(cols(h2), aug0(h2), src, dst, z, z0)
  h3 = _tc_layer(part3, p03, h2, Wl3, Wr3, b3, relu=False)
  return _tc_pool(h3, batch3, Wlin, blin)


# speedup vs baseline: 2.6858x; 1.3425x over previous
import jax
import jax.numpy as jnp
from jax import lax
from jax.experimental import pallas as pl
from jax.experimental.pallas import tpu as pltpu
from jax.experimental.pallas import tpu_sc as plsc

N = 10000
E = 320000
D = 128
G = 128
NC = 2
NS = 16
CW = D // NS
CW0 = 2 * CW
EH = E // NC
C = 128
NBUF = 4
NIB = 10
NCB = EH // (C * NIB)
_B = 1000
_NBLK = N // _B

_PREC = lax.Precision.HIGHEST


def _make_sc_edge():
  mesh = plsc.VectorSubcoreMesh(core_axis_name="c", subcore_axis_name="s",
                                num_cores=NC, num_subcores=NS)
  out_type = (jax.ShapeDtypeStruct((NC, NS - 1, N, CW), jnp.float32),
              jax.ShapeDtypeStruct((NC, N, CW0), jnp.float32))
  scratch = [
      pltpu.VMEM((NCB, C), jnp.int32),
      pltpu.VMEM((NCB, C), jnp.int32),
      pltpu.VMEM((NBUF, C, CW), jnp.float32),
      pltpu.VMEM((NBUF, C, CW0), jnp.float32),
      pltpu.VMEM_SHARED((NS - 1, N, CW), jnp.float32),
      pltpu.VMEM_SHARED((N, CW0), jnp.float32),
      tuple(pltpu.SemaphoreType.DMA for _ in range(NBUF)),
      tuple(pltpu.SemaphoreType.DMA for _ in range(NBUF)),
  ]

  def body(y_hbm, y0_hbm, src_hbm, dst_hbm, z_hbm, z0_hbm,
           part_hbm, part0_hbm, sidx, didx, rows, rows0, acc, acc0,
           gsem, ssem):
    c = lax.axis_index("c")
    s = lax.axis_index("s")

    @pl.when(s == 0)
    def _():
      pltpu.sync_copy(z0_hbm, acc0)

    @pl.when(s > 0)
    def _():
      pltpu.sync_copy(z_hbm, acc.at[s - 1])

    myacc = (acc0, acc.at[s - 1])
    ybase = (y0_hbm, y_hbm.at[s - 1])
    rbuf = (rows0, rows)

    def gissue(k, p):
      @pl.when(s == 0)
      def _():
        pltpu.async_copy(ybase[0].at[sidx.at[k]], rbuf[0].at[p], gsem[p])

      @pl.when(s > 0)
      def _():
        pltpu.async_copy(ybase[1].at[sidx.at[k]], rbuf[1].at[p], gsem[p])

    def gwait(k, p):
      @pl.when(s == 0)
      def _():
        pltpu.make_async_copy(ybase[0].at[sidx.at[k]], rbuf[0].at[p],
                              gsem[p]).wait()

      @pl.when(s > 0)
      def _():
        pltpu.make_async_copy(ybase[1].at[sidx.at[k]], rbuf[1].at[p],
                              gsem[p]).wait()

    def sissue(k, p):
      @pl.when(s == 0)
      def _():
        pltpu.async_copy(rbuf[0].at[p], myacc[0].at[didx.at[k]], ssem[p],
                         add=True)

      @pl.when(s > 0)
      def _():
        pltpu.async_copy(rbuf[1].at[p], myacc[1].at[didx.at[k]], ssem[p],
                         add=True)

    def swait(k, p):
      @pl.when(s == 0)
      def _():
        pltpu.make_async_copy(rbuf[0].at[p], myacc[0].at[didx.at[k]],
                              ssem[p]).wait()

      @pl.when(s > 0)
      def _():
        pltpu.make_async_copy(rbuf[1].at[p], myacc[1].at[didx.at[k]],
                              ssem[p]).wait()

    def block(b, carry):
      pltpu.sync_copy(src_hbm.at[c, b], sidx)
      pltpu.sync_copy(dst_hbm.at[c, b], didx)

      for j in range(NBUF - 1):
        gissue(j, j)

      def chunk(k, carry2):
        for p in range(NBUF):
          @pl.when(lax.rem(k, NBUF) == p)
          def _():
            q = (p + NBUF - 1) % NBUF
            @pl.when(k + NBUF - 1 < NCB)
            def _():
              @pl.when(k >= 1)
              def _():
                swait(k - 1, q)
              gissue(k + NBUF - 1, q)
            gwait(k, p)
            sissue(k, p)
        return carry2

      lax.fori_loop(0, NCB, chunk, 0)
      for j in range(NBUF):
        swait(NCB - NBUF + j, (NCB - NBUF + j) % NBUF)
      return carry

    lax.fori_loop(0, NIB, block, 0)

    @pl.when(s == 0)
    def _():
      pltpu.sync_copy(acc0, part0_hbm.at[c])

    @pl.when(s > 0)
    def _():
      pltpu.sync_copy(acc.at[s - 1], part_hbm.at[c, s - 1])

  return pl.kernel(body, out_type=out_type, mesh=mesh, scratch_types=scratch,
                   compiler_params=pltpu.CompilerParams(
                       use_tc_tiling_on_sc=False))


def _tc_layer(part, part0, h, Wl, Wr, b, *, relu):
  dw_out = Wr.shape[-1]
  hw = h.shape[-1]

  def body(part_ref, part0_ref, h_ref, Wl_ref, Wr_ref, b_ref, out_ref):
    p0 = part0_ref[0] + part0_ref[1]
    deg = jnp.maximum(p0[:, CW], 1.0)
    acc = jnp.dot(p0[:, :CW], Wl_ref[0:CW, :], precision=_PREC,
                  preferred_element_type=jnp.float32)
    for s in range(1, NS):
      psum = part_ref[0, s - 1] + part_ref[1, s - 1]
      acc = acc + jnp.dot(psum, Wl_ref[s * CW:(s + 1) * CW, :],
                          precision=_PREC,
                          preferred_element_type=jnp.float32)
    acc = acc / deg[:, None]
    acc = acc + jnp.dot(h_ref[...], Wr_ref[...], precision=_PREC,
                        preferred_element_type=jnp.float32) + b_ref[...]
    if relu:
      acc = jnp.maximum(acc, 0.0)
    out_ref[...] = acc

  return pl.pallas_call(
      body,
      grid=(_NBLK,),
      in_specs=[
          pl.BlockSpec((NC, NS - 1, _B, CW), lambda i: (0, 0, i, 0)),
          pl.BlockSpec((NC, _B, CW0), lambda i: (0, i, 0)),
          pl.BlockSpec((_B, hw), lambda i: (i, 0)),
          pl.BlockSpec(Wl.shape, lambda i: (0, 0)),
          pl.BlockSpec(Wr.shape, lambda i: (0, 0)),
          pl.BlockSpec((1, dw_out), lambda i: (0, 0)),
      ],
      out_specs=pl.BlockSpec((_B, dw_out), lambda i: (i, 0)),
      out_shape=jax.ShapeDtypeStruct((N, dw_out), jnp.float32),
  )(part, part0, h, Wl, Wr, b.reshape(1, dw_out))


def _tc_pool(h, batch3, Wlin, blin):
  dw = h.shape[-1]
  ow = Wlin.shape[-1]

  def body(h_ref, b_ref, Wlin_ref, blin_ref, out_ref, acc_s, acc_c):
    i = pl.program_id(0)

    @pl.when(i == 0)
    def _():
      acc_s[...] = jnp.zeros_like(acc_s)
      acc_c[...] = jnp.zeros_like(acc_c)

    bv = b_ref[0, 0, :]
    gi = lax.broadcasted_iota(jnp.int32, (G, _B), 0)
    mask = (bv[None, :] == gi).astype(jnp.float32)
    acc_s[...] += jnp.dot(mask, h_ref[...], precision=_PREC,
                          preferred_element_type=jnp.float32)
    acc_c[...] += jnp.sum(mask, axis=1, keepdims=True)

    @pl.when(i == _NBLK - 1)
    def _():
      pooled = acc_s[...] / jnp.maximum(acc_c[...], 1.0)
      out_ref[...] = jnp.dot(pooled, Wlin_ref[...], precision=_PREC,
                             preferred_element_type=jnp.float32) + blin_ref[...]

  return pl.pallas_call(
      body,
      grid=(_NBLK,),
      in_specs=[
          pl.BlockSpec((_B, dw), lambda i: (i, 0)),
          pl.BlockSpec((1, 1, _B), lambda i: (i, 0, 0)),
          pl.BlockSpec(Wlin.shape, lambda i: (0, 0)),
          pl.BlockSpec((1, ow), lambda i: (0, 0)),
      ],
      out_specs=pl.BlockSpec((G, ow), lambda i: (0, 0)),
      out_shape=jax.ShapeDtypeStruct((G, ow), jnp.float32),
      scratch_shapes=[pltpu.VMEM((G, dw), jnp.float32),
                      pltpu.VMEM((G, 1), jnp.float32)],
  )(h, batch3, Wlin, blin.reshape(1, ow))


def kernel(x, edge_index, batch, Wl1, Wr1, b1, Wl2, Wr2, b2,
           Wl3, Wr3, b3, Wlin, blin):
  src = edge_index[0].astype(jnp.int32).reshape(NC, NIB, NCB, C)
  dst = edge_index[1].astype(jnp.int32).reshape(NC, NIB, NCB, C)
  batch3 = batch.astype(jnp.int32).reshape(_NBLK, 1, _B)
  z = jnp.zeros((N, CW), jnp.float32)
  z0 = jnp.zeros((N, CW0), jnp.float32)
  onecol = jnp.concatenate([jnp.ones((N, 1), jnp.float32),
                            jnp.zeros((N, CW - 1), jnp.float32)], axis=1)

  sc = _make_sc_edge()

  def cols(y):
    return y.reshape(N, NS, CW).transpose(1, 0, 2)[1:]

  def aug0(y):
    return jnp.concatenate([y[:, :CW], onecol], axis=1)

  part1, p01 = sc(cols(x), aug0(x), src, dst, z, z0)
  h1 = _tc_layer(part1, p01, x, Wl1, Wr1, b1, relu=True)
  part2, p02 = sc(cols(h1), aug0(h1), src, dst, z, z0)
  h2 = _tc_layer(part2, p02, h1, Wl2, Wr2, b2, relu=True)
  part3, p03 = sc(cols(h2), aug0(h2), src, dst, z, z0)
  h3 = _tc_layer(part3, p03, h2, Wl3, Wr3, b3, relu=False)
  return _tc_pool(h3, batch3, Wlin, blin)


# speedup vs baseline: 2.8841x; 1.0739x over previous
import jax
import jax.numpy as jnp
from jax import lax
from jax.experimental import pallas as pl
from jax.experimental.pallas import tpu as pltpu
from jax.experimental.pallas import tpu_sc as plsc

N = 10000
E = 320000
D = 128
G = 128
NC = 2
NS = 16
CW = D // NS
CW0 = 2 * CW
EH = E // NC
C = 128
NBUF = 4
NIB = 10
NCB = EH // (C * NIB)
_B = 1000
_NBLK = N // _B

_PREC = lax.Precision.HIGHEST


def _make_sc_edge():
  mesh = plsc.VectorSubcoreMesh(core_axis_name="c", subcore_axis_name="s",
                                num_cores=NC, num_subcores=NS)
  out_type = (jax.ShapeDtypeStruct((NC, NS - 1, N, CW), jnp.float32),
              jax.ShapeDtypeStruct((NC, N, CW0), jnp.float32))
  scratch = [
      pltpu.VMEM((NCB, C), jnp.int32),
      pltpu.VMEM((NCB, C), jnp.int32),
      pltpu.VMEM((NBUF, C, CW), jnp.float32),
      pltpu.VMEM((NBUF, C, CW0), jnp.float32),
      pltpu.VMEM_SHARED((NS - 1, N, CW), jnp.float32),
      pltpu.VMEM_SHARED((N, CW0), jnp.float32),
      tuple(pltpu.SemaphoreType.DMA for _ in range(NBUF)),
      tuple(pltpu.SemaphoreType.DMA for _ in range(NBUF)),
  ]

  def body(y_hbm, y0_hbm, src_hbm, dst_hbm, z_hbm, z0_hbm,
           part_hbm, part0_hbm, sidx, didx, rows, rows0, acc, acc0,
           gsem, ssem):
    c = lax.axis_index("c")
    s = lax.axis_index("s")

    @pl.when(s == 0)
    def _():
      pltpu.sync_copy(z0_hbm, acc0)

    @pl.when(s > 0)
    def _():
      pltpu.sync_copy(z_hbm, acc.at[s - 1])

    myacc = (acc0, acc.at[s - 1])
    ybase = (y0_hbm, y_hbm.at[s - 1])
    rbuf = (rows0, rows)

    def gissue(k, p):
      @pl.when(s == 0)
      def _():
        pltpu.async_copy(ybase[0].at[sidx.at[k]], rbuf[0].at[p], gsem[p])

      @pl.when(s > 0)
      def _():
        pltpu.async_copy(ybase[1].at[sidx.at[k]], rbuf[1].at[p], gsem[p])

    def gwait(k, p):
      @pl.when(s == 0)
      def _():
        pltpu.make_async_copy(ybase[0].at[sidx.at[k]], rbuf[0].at[p],
                              gsem[p]).wait()

      @pl.when(s > 0)
      def _():
        pltpu.make_async_copy(ybase[1].at[sidx.at[k]], rbuf[1].at[p],
                              gsem[p]).wait()

    def sissue(k, p):
      @pl.when(s == 0)
      def _():
        pltpu.async_copy(rbuf[0].at[p], myacc[0].at[didx.at[k]], ssem[p],
                         add=True)

      @pl.when(s > 0)
      def _():
        pltpu.async_copy(rbuf[1].at[p], myacc[1].at[didx.at[k]], ssem[p],
                         add=True)

    def swait(k, p):
      @pl.when(s == 0)
      def _():
        pltpu.make_async_copy(rbuf[0].at[p], myacc[0].at[didx.at[k]],
                              ssem[p]).wait()

      @pl.when(s > 0)
      def _():
        pltpu.make_async_copy(rbuf[1].at[p], myacc[1].at[didx.at[k]],
                              ssem[p]).wait()

    def block(b, carry):
      pltpu.sync_copy(src_hbm.at[c, b], sidx)
      pltpu.sync_copy(dst_hbm.at[c, b], didx)

      for j in range(NBUF - 1):
        gissue(j, j)

      def chunk(k, carry2):
        for p in range(NBUF):
          @pl.when(lax.rem(k, NBUF) == p)
          def _():
            q = (p + NBUF - 1) % NBUF
            @pl.when(k + NBUF - 1 < NCB)
            def _():
              @pl.when(k >= 1)
              def _():
                swait(k - 1, q)
              gissue(k + NBUF - 1, q)
            gwait(k, p)
            sissue(k, p)
        return carry2

      lax.fori_loop(0, NCB, chunk, 0)
      for j in range(NBUF):
        swait(NCB - NBUF + j, (NCB - NBUF + j) % NBUF)
      return carry

    lax.fori_loop(0, NIB, block, 0)

    @pl.when(s == 0)
    def _():
      pltpu.sync_copy(acc0, part0_hbm.at[c])

    @pl.when(s > 0)
    def _():
      pltpu.sync_copy(acc.at[s - 1], part_hbm.at[c, s - 1])

  return pl.kernel(body, out_type=out_type, mesh=mesh, scratch_types=scratch,
                   compiler_params=pltpu.CompilerParams(
                       use_tc_tiling_on_sc=False))


def _layer_acc(part_ref, part0_ref, h_ref, Wl_ref, Wr_ref, b_ref, relu):
  p0 = part0_ref[0] + part0_ref[1]
  deg = jnp.maximum(p0[:, CW], 1.0)
  acc = jnp.dot(p0[:, :CW], Wl_ref[0:CW, :], precision=_PREC,
                preferred_element_type=jnp.float32)
  for s in range(1, NS):
    psum = part_ref[0, s - 1] + part_ref[1, s - 1]
    acc = acc + jnp.dot(psum, Wl_ref[s * CW:(s + 1) * CW, :],
                        precision=_PREC,
                        preferred_element_type=jnp.float32)
  acc = acc / deg[:, None]
  acc = acc + jnp.dot(h_ref[...], Wr_ref[...], precision=_PREC,
                      preferred_element_type=jnp.float32) + b_ref[...]
  if relu:
    acc = jnp.maximum(acc, 0.0)
  return acc


def _tc_layer(part, part0, h, Wl, Wr, b):
  dw_out = Wr.shape[-1]
  hw = h.shape[-1]

  def body(part_ref, part0_ref, h_ref, Wl_ref, Wr_ref, b_ref,
           out_ref, yc_ref, y0_ref):
    acc = _layer_acc(part_ref, part0_ref, h_ref, Wl_ref, Wr_ref, b_ref, True)
    out_ref[...] = acc
    for j in range(NS - 1):
      yc_ref[j] = acc[:, (j + 1) * CW:(j + 2) * CW]
    y0_ref[...] = jnp.concatenate(
        [acc[:, :CW], jnp.ones((_B, 1), jnp.float32),
         jnp.zeros((_B, CW0 - CW - 1), jnp.float32)], axis=1)

  return pl.pallas_call(
      body,
      grid=(_NBLK,),
      in_specs=[
          pl.BlockSpec((NC, NS - 1, _B, CW), lambda i: (0, 0, i, 0)),
          pl.BlockSpec((NC, _B, CW0), lambda i: (0, i, 0)),
          pl.BlockSpec((_B, hw), lambda i: (i, 0)),
          pl.BlockSpec(Wl.shape, lambda i: (0, 0)),
          pl.BlockSpec(Wr.shape, lambda i: (0, 0)),
          pl.BlockSpec((1, dw_out), lambda i: (0, 0)),
      ],
      out_specs=[
          pl.BlockSpec((_B, dw_out), lambda i: (i, 0)),
          pl.BlockSpec((NS - 1, _B, CW), lambda i: (0, i, 0)),
          pl.BlockSpec((_B, CW0), lambda i: (i, 0)),
      ],
      out_shape=[
          jax.ShapeDtypeStruct((N, dw_out), jnp.float32),
          jax.ShapeDtypeStruct((NS - 1, N, CW), jnp.float32),
          jax.ShapeDtypeStruct((N, CW0), jnp.float32),
      ],
  )(part, part0, h, Wl, Wr, b.reshape(1, dw_out))


def _tc_layer3_pool(part, part0, h, Wl, Wr, b, batch3, Wlin, blin):
  dw_out = Wr.shape[-1]
  hw = h.shape[-1]
  ow = Wlin.shape[-1]

  def body(part_ref, part0_ref, h_ref, Wl_ref, Wr_ref, b_ref, batch_ref,
           Wlin_ref, blin_ref, out_ref, acc_s, acc_c):
    i = pl.program_id(0)
    acc = _layer_acc(part_ref, part0_ref, h_ref, Wl_ref, Wr_ref, b_ref, False)

    @pl.when(i == 0)
    def _():
      acc_s[...] = jnp.zeros_like(acc_s)
      acc_c[...] = jnp.zeros_like(acc_c)

    bv = batch_ref[0, 0, :]
    gi = lax.broadcasted_iota(jnp.int32, (G, _B), 0)
    mask = (bv[None, :] == gi).astype(jnp.float32)
    acc_s[...] += jnp.dot(mask, acc, precision=_PREC,
                          preferred_element_type=jnp.float32)
    acc_c[...] += jnp.sum(mask, axis=1, keepdims=True)

    @pl.when(i == _NBLK - 1)
    def _():
      pooled = acc_s[...] / jnp.maximum(acc_c[...], 1.0)
      out_ref[...] = jnp.dot(pooled, Wlin_ref[...], precision=_PREC,
                             preferred_element_type=jnp.float32) + blin_ref[...]

  return pl.pallas_call(
      body,
      grid=(_NBLK,),
      in_specs=[
          pl.BlockSpec((NC, NS - 1, _B, CW), lambda i: (0, 0, i, 0)),
          pl.BlockSpec((NC, _B, CW0), lambda i: (0, i, 0)),
          pl.BlockSpec((_B, hw), lambda i: (i, 0)),
          pl.BlockSpec(Wl.shape, lambda i: (0, 0)),
          pl.BlockSpec(Wr.shape, lambda i: (0, 0)),
          pl.BlockSpec((1, dw_out), lambda i: (0, 0)),
          pl.BlockSpec((1, 1, _B), lambda i: (i, 0, 0)),
          pl.BlockSpec(Wlin.shape, lambda i: (0, 0)),
          pl.BlockSpec((1, ow), lambda i: (0, 0)),
      ],
      out_specs=pl.BlockSpec((G, ow), lambda i: (0, 0)),
      out_shape=jax.ShapeDtypeStruct((G, ow), jnp.float32),
      scratch_shapes=[pltpu.VMEM((G, dw_out), jnp.float32),
                      pltpu.VMEM((G, 1), jnp.float32)],
  )(part, part0, h, Wl, Wr, b.reshape(1, dw_out), batch3,
    Wlin, blin.reshape(1, ow))


def kernel(x, edge_index, batch, Wl1, Wr1, b1, Wl2, Wr2, b2,
           Wl3, Wr3, b3, Wlin, blin):
  src = edge_index[0].astype(jnp.int32).reshape(NC, NIB, NCB, C)
  dst = edge_index[1].astype(jnp.int32).reshape(NC, NIB, NCB, C)
  batch3 = batch.astype(jnp.int32).reshape(_NBLK, 1, _B)
  z = jnp.zeros((N, CW), jnp.float32)
  z0 = jnp.zeros((N, CW0), jnp.float32)
  onecol = jnp.concatenate([jnp.ones((N, 1), jnp.float32),
                            jnp.zeros((N, CW - 1), jnp.float32)], axis=1)

  sc = _make_sc_edge()

  def cols(y):
    return y.reshape(N, NS, CW).transpose(1, 0, 2)[1:]

  def aug0(y):
    return jnp.concatenate([y[:, :CW], onecol], axis=1)

  part1, p01 = sc(cols(x), aug0(x), src, dst, z, z0)
  h1, y1, y01 = _tc_layer(part1, p01, x, Wl1, Wr1, b1)
  part2, p02 = sc(y1, y01, src, dst, z, z0)
  h2, y2, y02 = _tc_layer(part2, p02, h1, Wl2, Wr2, b2)
  part3, p03 = sc(y2, y02, src, dst, z, z0)
  return _tc_layer3_pool(part3, p03, h2, Wl3, Wr3, b3, batch3, Wlin, blin)
